# baseline (speedup 1.0000x reference)
import numpy as np

import jax
import jax.numpy as jnp
from jax import lax
from jax.experimental import pallas as pl
from jax.experimental.pallas import tpu as pltpu

_EPS = 1e-5
_TM = 512
_VMEM_LIMIT = 60 * 1024 * 1024


def _round_up(x, k):
    return (x + k - 1) // k * k


def _build_x3(xt, tm, H, W):
    h2 = H // 2
    xe = xt[:, 0]
    xo = xt[:, 1]
    z = jnp.zeros((tm, 1, W), jnp.float32)
    dn = jnp.concatenate([z, xo[:, :h2 - 1]], axis=1)
    up = jnp.concatenate([xe[:, 1:], z], axis=1)

    def sec(a, b):
        return jnp.concatenate([a[:, None], b[:, None]], axis=1).reshape(tm * H, W)

    s0 = sec(dn, xe)
    s1 = xt.reshape(tm * H, W)
    s2 = sec(xo, up)
    ones = jnp.ones((tm * H, 1), jnp.float32)
    return jnp.concatenate([s0, s1, s2, ones], axis=1)


def _gram_kernel(x_ref, gram_ref, cs_ref):
    shp = x_ref.shape
    tm = shp[0] * shp[1] * shp[2] if len(shp) == 5 else shp[0]
    H, W = shp[-2], shp[-1]
    xw = x_ref[...].reshape(tm, H * W)
    gram = lax.dot_general(xw, xw, (((0,), (0,)), ((), ())),
                           preferred_element_type=jnp.float32)
    gram_ref[...] = gram[None]
    cs_ref[...] = jnp.sum(xw, axis=0).reshape(1, 1, H * W)


def _main_kernel(xt_ref, b_ref, fc1_ref, vec_ref, out_ref):
    tm, _, h2, W = xt_ref.shape
    H = 2 * h2
    C = fc1_ref.shape[0]
    w2 = W // 2
    x3 = _build_x3(xt_ref[...], tm, H, W)
    y = jnp.dot(x3, b_ref[...], preferred_element_type=jnp.float32)
    y = y.reshape(tm, 2, h2, W * C)
    v = jnp.maximum(y[:, 0], y[:, 1]).reshape(tm * h2, W * C)
    half = w2 * C
    z = jnp.maximum(jnp.maximum(v[:, :half], v[:, half:]), 0.0)
    while z.shape[1] > C:
        hw = z.shape[1] // 2
        z = z[:, :hw] + z[:, hw:]
    feat = jnp.sum(z.reshape(tm, h2, C), axis=1)
    vecs = vec_ref[...]
    h = jnp.dot(feat, fc1_ref[...], preferred_element_type=jnp.float32) + vecs[0:1, :]
    h = jnp.maximum(h, 0.0)
    logit = jnp.sum(h * vecs[1:2, :], axis=-1, keepdims=True) + vecs[2:3, 0:1]
    out_ref[...] = (1.0 / (1.0 + jnp.exp(-logit))).reshape(1, tm, 1)


def kernel(x, conv_w, conv_b, gamma, beta, fc1_w, fc1_b, fc2_w, fc2_b):
    d0, d1, J, H, W = x.shape
    assert H % 2 == 0 and W % 2 == 0
    M = d0 * d1 * J
    C = conv_w.shape[-1]
    K3 = 3 * W + 1

    xm = x.reshape(M, H, W).astype(jnp.float32)
    w9 = conv_w.reshape(9, C).astype(jnp.float32)

    tm = min(_TM, _round_up(M, 8))
    Mp = _round_up(M, tm)
    nt = Mp // tm
    xp = jnp.pad(xm, ((0, Mp - M), (0, 0), (0, 0)))
    xr = xp.reshape(Mp, H // 2, 2, W).transpose(0, 2, 1, 3)

    HW = H * W
    if M % tm == 0 and (d1 * J) % 8 == 0 and tm % (d1 * J) == 0:
        td = tm // (d1 * J)
        nt1 = d0 // td
        in_spec1 = pl.BlockSpec((td, d1, J, H, W), lambda i: (i, 0, 0, 0, 0))
        x_in = x.astype(jnp.float32)
    else:
        nt1 = nt
        in_spec1 = pl.BlockSpec((tm, H, W), lambda i: (i, 0, 0))
        x_in = xp
    gram_t, cs_t = pl.pallas_call(
        _gram_kernel,
        out_shape=[jax.ShapeDtypeStruct((nt1, HW, HW), jnp.float32),
                   jax.ShapeDtypeStruct((nt1, 1, HW), jnp.float32)],
        grid=(nt1,),
        in_specs=[in_spec1],
        out_specs=[pl.BlockSpec((1, HW, HW), lambda i: (i, 0, 0)),
                   pl.BlockSpec((1, 1, HW), lambda i: (i, 0, 0))],
        compiler_params=pltpu.CompilerParams(
            dimension_semantics=("parallel",),
            vmem_limit_bytes=_VMEM_LIMIT),
    )(x_in)
    G4 = jnp.sum(gram_t, axis=0).reshape(H, W, H, W)
    Scol = jnp.sum(cs_t, axis=0).reshape(H, W)

    U = np.zeros((9, H, H), np.float32)
    Wt = np.zeros((9, W, W), np.float32)
    Su = np.zeros((3, H), np.float32)
    Sw = np.zeros((3, W), np.float32)
    for ka in range(3):
        for kb in range(3):
            for h in range(H):
                u, v = h + ka - 1, h + kb - 1
                if 0 <= u < H and 0 <= v < H:
                    U[ka * 3 + kb, u, v] += 1.0
            for w in range(W):
                a, b = w + ka - 1, w + kb - 1
                if 0 <= a < W and 0 <= b < W:
                    Wt[ka * 3 + kb, a, b] += 1.0
        for h in range(H):
            if 0 <= h + ka - 1 < H:
                Su[ka, h + ka - 1] += 1.0
        for w in range(W):
            if 0 <= w + ka - 1 < W:
                Sw[ka, w + ka - 1] += 1.0
    Gq = jnp.einsum("uavb,quv,sab->qs", G4, U, Wt)
    G = Gq.reshape(3, 3, 3, 3).transpose(0, 2, 1, 3).reshape(9, 9)
    S = jnp.einsum("ua,qu,sa->qs", Scol, Su, Sw).reshape(9)

    count = float(M * H * W)
    mean = jnp.dot(S, w9) / count
    ssq = jnp.einsum("kc,kl,lc->c", w9, G, w9)
    var = jnp.maximum(ssq / count - mean * mean, 0.0)
    scale = gamma * lax.rsqrt(var + _EPS)
    shift = beta - scale * mean
    pool_inv = 1.0 / ((H // 2) * (W // 2))
    sf = scale * pool_inv
    hf = shift * pool_inv

    place = np.zeros((9, K3 - 1, W), np.float32)
    for ky in range(3):
        for kx in range(3):
            for w in range(W):
                wp = w + kx - 1
                if 0 <= wp < W:
                    place[ky * 3 + kx, ky * W + wp, w] = 1.0
    w9s = w9 * sf[None, :]
    Bwc = jnp.einsum("trw,tc->rwc", place, w9s)
    Bmain = Bwc.reshape(K3 - 1, W // 2, 2, C).transpose(0, 2, 1, 3).reshape(K3 - 1, W * C)
    shift_row = jnp.broadcast_to(hf[None, :], (W, C)).reshape(1, W * C)
    B = jnp.concatenate([Bmain, shift_row], axis=0)

    vecs = jnp.stack([fc1_b, fc2_w.reshape(-1),
                      jnp.full((C,), fc2_b[0], jnp.float32)], axis=0)

    scores = pl.pallas_call(
        _main_kernel,
        out_shape=jax.ShapeDtypeStruct((nt, tm, 1), jnp.float32),
        grid=(nt,),
        in_specs=[pl.BlockSpec((tm, 2, H // 2, W), lambda i: (i, 0, 0, 0)),
                  pl.BlockSpec((K3, W * C), lambda i: (0, 0)),
                  pl.BlockSpec((C, C), lambda i: (0, 0)),
                  pl.BlockSpec((3, C), lambda i: (0, 0))],
        out_specs=pl.BlockSpec((1, tm, 1), lambda i: (i, 0, 0)),
        compiler_params=pltpu.CompilerParams(
            dimension_semantics=("parallel",),
            vmem_limit_bytes=_VMEM_LIMIT),
    )(xr, B, fc1_w, vecs)

    return scores.reshape(Mp, 1)[:M].reshape(d0 * d1, J, 1)


# speedup vs baseline: 1.1734x; 1.1734x over previous
import numpy as np

import jax
import jax.numpy as jnp
from jax import lax
from jax.experimental import pallas as pl
from jax.experimental.pallas import tpu as pltpu

_EPS = 1e-5
_TM = 512
_VMEM_LIMIT = 60 * 1024 * 1024


def _round_up(x, k):
    return (x + k - 1) // k * k


def _build_x3(xt, tm, H, W):
    h2 = H // 2
    xe = xt[:, 0]
    xo = xt[:, 1]
    z = jnp.zeros((tm, 1, W), jnp.float32)
    dn = jnp.concatenate([z, xo[:, :h2 - 1]], axis=1)
    up = jnp.concatenate([xe[:, 1:], z], axis=1)

    def sec(a, b):
        return jnp.concatenate([a[:, None], b[:, None]], axis=1).reshape(tm * H, W)

    s0 = sec(dn, xe)
    s1 = xt.reshape(tm * H, W)
    s2 = sec(xo, up)
    ones = jnp.ones((tm * H, 1), jnp.float32)
    return jnp.concatenate([s0, s1, s2, ones], axis=1)


def _gram_kernel(x_ref, gram_ref, cs_ref):
    shp = x_ref.shape
    tm = shp[0] * shp[1] * shp[2] if len(shp) == 5 else shp[0]
    H, W = shp[-2], shp[-1]
    xw = x_ref[...].reshape(tm, H * W)
    gram = lax.dot_general(xw, xw, (((0,), (0,)), ((), ())),
                           preferred_element_type=jnp.float32)
    gram_ref[...] = gram[None]
    cs_ref[...] = jnp.sum(xw, axis=0).reshape(1, 1, H * W)


def _main_kernel(xt_ref, b_ref, fc1_ref, vec_ref, out_ref):
    tm, _, h2, W = xt_ref.shape
    H = 2 * h2
    C = fc1_ref.shape[0]
    w2 = W // 2
    x3 = _build_x3(xt_ref[...], tm, H, W)
    y = jnp.dot(x3, b_ref[...], preferred_element_type=jnp.float32)
    y = y.reshape(tm, 2, h2, W * C)
    v = jnp.maximum(y[:, 0], y[:, 1]).reshape(tm * h2, W * C)
    half = w2 * C
    z = jnp.maximum(jnp.maximum(v[:, :half], v[:, half:]), 0.0)
    while z.shape[1] > C:
        hw = z.shape[1] // 2
        z = z[:, :hw] + z[:, hw:]
    feat = jnp.sum(z.reshape(tm, h2, C), axis=1)
    vecs = vec_ref[...]
    h = jnp.dot(feat, fc1_ref[...], preferred_element_type=jnp.float32) + vecs[0:1, :]
    h = jnp.maximum(h, 0.0)
    logit = jnp.sum(h * vecs[1:2, :], axis=-1, keepdims=True) + vecs[2:3, 0:1]
    out_ref[...] = (1.0 / (1.0 + jnp.exp(-logit))).reshape(1, tm, 1)


def kernel(x, conv_w, conv_b, gamma, beta, fc1_w, fc1_b, fc2_w, fc2_b):
    d0, d1, J, H, W = x.shape
    assert H % 2 == 0 and W % 2 == 0
    M = d0 * d1 * J
    C = conv_w.shape[-1]
    K3 = 3 * W + 1

    xm = x.reshape(M, H, W).astype(jnp.float32)
    w9 = conv_w.reshape(9, C).astype(jnp.float32)

    tm = min(_TM, _round_up(M, 8))
    Mp = _round_up(M, tm)
    nt = Mp // tm
    xp = jnp.pad(xm, ((0, Mp - M), (0, 0), (0, 0)))
    xr = xp.reshape(Mp, H // 2, 2, W).transpose(0, 2, 1, 3)

    HW = H * W
    nt1 = nt
    in_spec1 = pl.BlockSpec((tm, H, W), lambda i: (i, 0, 0))
    x_in = xp
    gram_t, cs_t = pl.pallas_call(
        _gram_kernel,
        out_shape=[jax.ShapeDtypeStruct((nt1, HW, HW), jnp.float32),
                   jax.ShapeDtypeStruct((nt1, 1, HW), jnp.float32)],
        grid=(nt1,),
        in_specs=[in_spec1],
        out_specs=[pl.BlockSpec((1, HW, HW), lambda i: (i, 0, 0)),
                   pl.BlockSpec((1, 1, HW), lambda i: (i, 0, 0))],
        compiler_params=pltpu.CompilerParams(
            dimension_semantics=("parallel",),
            vmem_limit_bytes=_VMEM_LIMIT),
    )(x_in)
    G4 = jnp.sum(gram_t, axis=0).reshape(H, W, H, W)
    Scol = jnp.sum(cs_t, axis=0).reshape(H, W)

    U = np.zeros((9, H, H), np.float32)
    Wt = np.zeros((9, W, W), np.float32)
    Su = np.zeros((3, H), np.float32)
    Sw = np.zeros((3, W), np.float32)
    for ka in range(3):
        for kb in range(3):
            for h in range(H):
                u, v = h + ka - 1, h + kb - 1
                if 0 <= u < H and 0 <= v < H:
                    U[ka * 3 + kb, u, v] += 1.0
            for w in range(W):
                a, b = w + ka - 1, w + kb - 1
                if 0 <= a < W and 0 <= b < W:
                    Wt[ka * 3 + kb, a, b] += 1.0
        for h in range(H):
            if 0 <= h + ka - 1 < H:
                Su[ka, h + ka - 1] += 1.0
        for w in range(W):
            if 0 <= w + ka - 1 < W:
                Sw[ka, w + ka - 1] += 1.0
    Gq = jnp.einsum("uavb,quv,sab->qs", G4, U, Wt)
    G = Gq.reshape(3, 3, 3, 3).transpose(0, 2, 1, 3).reshape(9, 9)
    S = jnp.einsum("ua,qu,sa->qs", Scol, Su, Sw).reshape(9)

    count = float(M * H * W)
    mean = jnp.dot(S, w9) / count
    ssq = jnp.einsum("kc,kl,lc->c", w9, G, w9)
    var = jnp.maximum(ssq / count - mean * mean, 0.0)
    scale = gamma * lax.rsqrt(var + _EPS)
    shift = beta - scale * mean
    pool_inv = 1.0 / ((H // 2) * (W // 2))
    sf = scale * pool_inv
    hf = shift * pool_inv

    place = np.zeros((9, K3 - 1, W), np.float32)
    for ky in range(3):
        for kx in range(3):
            for w in range(W):
                wp = w + kx - 1
                if 0 <= wp < W:
                    place[ky * 3 + kx, ky * W + wp, w] = 1.0
    w9s = w9 * sf[None, :]
    Bwc = jnp.einsum("trw,tc->rwc", place, w9s)
    Bmain = Bwc.reshape(K3 - 1, W // 2, 2, C).transpose(0, 2, 1, 3).reshape(K3 - 1, W * C)
    shift_row = jnp.broadcast_to(hf[None, :], (W, C)).reshape(1, W * C)
    B = jnp.concatenate([Bmain, shift_row], axis=0)

    vecs = jnp.stack([fc1_b, fc2_w.reshape(-1),
                      jnp.full((C,), fc2_b[0], jnp.float32)], axis=0)

    scores = pl.pallas_call(
        _main_kernel,
        out_shape=jax.ShapeDtypeStruct((nt, tm, 1), jnp.float32),
        grid=(nt,),
        in_specs=[pl.BlockSpec((tm, 2, H // 2, W), lambda i: (i, 0, 0, 0)),
                  pl.BlockSpec((K3, W * C), lambda i: (0, 0)),
                  pl.BlockSpec((C, C), lambda i: (0, 0)),
                  pl.BlockSpec((3, C), lambda i: (0, 0))],
        out_specs=pl.BlockSpec((1, tm, 1), lambda i: (i, 0, 0)),
        compiler_params=pltpu.CompilerParams(
            dimension_semantics=("parallel",),
            vmem_limit_bytes=_VMEM_LIMIT),
    )(xr, B, fc1_w, vecs)

    return scores.reshape(Mp, 1)[:M].reshape(d0 * d1, J, 1)


# speedup vs baseline: 1.1931x; 1.0168x over previous
import numpy as np

import jax
import jax.numpy as jnp
from jax import lax
from jax.experimental import pallas as pl
from jax.experimental.pallas import tpu as pltpu

_EPS = 1e-5
_TM = 512
_VMEM_LIMIT = 60 * 1024 * 1024


def _round_up(x, k):
    return (x + k - 1) // k * k


def _build_x3(xt, tm, H, W):
    h2 = H // 2
    xe = xt[:, 0]
    xo = xt[:, 1]
    z = jnp.zeros((tm, 1, W), jnp.float32)
    dn = jnp.concatenate([z, xo[:, :h2 - 1]], axis=1)
    up = jnp.concatenate([xe[:, 1:], z], axis=1)

    def sec(a, b):
        return jnp.concatenate([a[:, None], b[:, None]], axis=1).reshape(tm * H, W)

    s0 = sec(dn, xe)
    s1 = xt.reshape(tm * H, W)
    s2 = sec(xo, up)
    ones = jnp.ones((tm * H, 1), jnp.float32)
    return jnp.concatenate([s0, s1, s2, ones], axis=1)


def _gram_kernel(x_ref, gram_ref, cs_ref):
    tm, H, W = x_ref.shape
    xw = x_ref[...].reshape(tm, H * W)
    gram = lax.dot_general(xw, xw, (((0,), (0,)), ((), ())),
                           preferred_element_type=jnp.float32)[None]
    cs = jnp.sum(xw, axis=0).reshape(1, 1, H * W)
    i = pl.program_id(0)

    @pl.when(i == 0)
    def _():
        gram_ref[...] = gram
        cs_ref[...] = cs

    @pl.when(i > 0)
    def _():
        gram_ref[...] += gram
        cs_ref[...] += cs


def _main_kernel(xt_ref, b_ref, fc1_ref, vec_ref, out_ref):
    tm, _, h2, W = xt_ref.shape
    H = 2 * h2
    C = fc1_ref.shape[0]
    w2 = W // 2
    x3 = _build_x3(xt_ref[...], tm, H, W)
    y = jnp.dot(x3, b_ref[...], preferred_element_type=jnp.float32)
    y = y.reshape(tm, 2, h2, W * C)
    v = jnp.maximum(y[:, 0], y[:, 1]).reshape(tm * h2, W * C)
    half = w2 * C
    z = jnp.maximum(jnp.maximum(v[:, :half], v[:, half:]), 0.0)
    while z.shape[1] > C:
        hw = z.shape[1] // 2
        z = z[:, :hw] + z[:, hw:]
    feat = jnp.sum(z.reshape(tm, h2, C), axis=1)
    vecs = vec_ref[...]
    h = jnp.dot(feat, fc1_ref[...], preferred_element_type=jnp.float32) + vecs[0:1, :]
    h = jnp.maximum(h, 0.0)
    logit = jnp.sum(h * vecs[1:2, :], axis=-1, keepdims=True) + vecs[2:3, 0:1]
    out_ref[...] = (1.0 / (1.0 + jnp.exp(-logit))).reshape(1, tm, 1)


def kernel(x, conv_w, conv_b, gamma, beta, fc1_w, fc1_b, fc2_w, fc2_b):
    d0, d1, J, H, W = x.shape
    assert H % 2 == 0 and W % 2 == 0
    M = d0 * d1 * J
    C = conv_w.shape[-1]
    K3 = 3 * W + 1

    xm = x.reshape(M, H, W).astype(jnp.float32)
    w9 = conv_w.reshape(9, C).astype(jnp.float32)

    tm = min(_TM, _round_up(M, 8))
    Mp = _round_up(M, tm)
    nt = Mp // tm
    xp = jnp.pad(xm, ((0, Mp - M), (0, 0), (0, 0)))
    xr = xp.reshape(Mp, H // 2, 2, W).transpose(0, 2, 1, 3)

    HW = H * W
    gram_t, cs_t = pl.pallas_call(
        _gram_kernel,
        out_shape=[jax.ShapeDtypeStruct((1, HW, HW), jnp.float32),
                   jax.ShapeDtypeStruct((1, 1, HW), jnp.float32)],
        grid=(nt,),
        in_specs=[pl.BlockSpec((tm, H, W), lambda i: (i, 0, 0))],
        out_specs=[pl.BlockSpec((1, HW, HW), lambda i: (0, 0, 0)),
                   pl.BlockSpec((1, 1, HW), lambda i: (0, 0, 0))],
        compiler_params=pltpu.CompilerParams(
            dimension_semantics=("arbitrary",),
            vmem_limit_bytes=_VMEM_LIMIT),
    )(xp)
    G4 = gram_t.reshape(H, W, H, W)
    Scol = cs_t.reshape(H, W)

    U = np.zeros((9, H, H), np.float32)
    Wt = np.zeros((9, W, W), np.float32)
    Su = np.zeros((3, H), np.float32)
    Sw = np.zeros((3, W), np.float32)
    for ka in range(3):
        for kb in range(3):
            for h in range(H):
                u, v = h + ka - 1, h + kb - 1
                if 0 <= u < H and 0 <= v < H:
                    U[ka * 3 + kb, u, v] += 1.0
            for w in range(W):
                a, b = w + ka - 1, w + kb - 1
                if 0 <= a < W and 0 <= b < W:
                    Wt[ka * 3 + kb, a, b] += 1.0
        for h in range(H):
            if 0 <= h + ka - 1 < H:
                Su[ka, h + ka - 1] += 1.0
        for w in range(W):
            if 0 <= w + ka - 1 < W:
                Sw[ka, w + ka - 1] += 1.0
    Gq = jnp.einsum("uavb,quv,sab->qs", G4, U, Wt)
    G = Gq.reshape(3, 3, 3, 3).transpose(0, 2, 1, 3).reshape(9, 9)
    S = jnp.einsum("ua,qu,sa->qs", Scol, Su, Sw).reshape(9)

    count = float(M * H * W)
    mean = jnp.dot(S, w9) / count
    ssq = jnp.einsum("kc,kl,lc->c", w9, G, w9)
    var = jnp.maximum(ssq / count - mean * mean, 0.0)
    scale = gamma * lax.rsqrt(var + _EPS)
    shift = beta - scale * mean
    pool_inv = 1.0 / ((H // 2) * (W // 2))
    sf = scale * pool_inv
    hf = shift * pool_inv

    place = np.zeros((9, K3 - 1, W), np.float32)
    for ky in range(3):
        for kx in range(3):
            for w in range(W):
                wp = w + kx - 1
                if 0 <= wp < W:
                    place[ky * 3 + kx, ky * W + wp, w] = 1.0
    w9s = w9 * sf[None, :]
    Bwc = jnp.einsum("trw,tc->rwc", place, w9s)
    Bmain = Bwc.reshape(K3 - 1, W // 2, 2, C).transpose(0, 2, 1, 3).reshape(K3 - 1, W * C)
    shift_row = jnp.broadcast_to(hf[None, :], (W, C)).reshape(1, W * C)
    B = jnp.concatenate([Bmain, shift_row], axis=0)

    vecs = jnp.stack([fc1_b, fc2_w.reshape(-1),
                      jnp.full((C,), fc2_b[0], jnp.float32)], axis=0)

    scores = pl.pallas_call(
        _main_kernel,
        out_shape=jax.ShapeDtypeStruct((nt, tm, 1), jnp.float32),
        grid=(nt,),
        in_specs=[pl.BlockSpec((tm, 2, H // 2, W), lambda i: (i, 0, 0, 0)),
                  pl.BlockSpec((K3, W * C), lambda i: (0, 0)),
                  pl.BlockSpec((C, C), lambda i: (0, 0)),
                  pl.BlockSpec((3, C), lambda i: (0, 0))],
        out_specs=pl.BlockSpec((1, tm, 1), lambda i: (i, 0, 0)),
        compiler_params=pltpu.CompilerParams(
            dimension_semantics=("parallel",),
            vmem_limit_bytes=_VMEM_LIMIT),
    )(xr, B, fc1_w, vecs)

    return scores.reshape(Mp, 1)[:M].reshape(d0 * d1, J, 1)
